# baseline (speedup 1.0000x reference)
import functools

import jax
import jax.numpy as jnp
from jax import lax
from jax.experimental import pallas as pl
from jax.experimental.pallas import tpu as pltpu
from jax.experimental.pallas import tpu_sc as plsc

N = 10000
E = 320000
D = 128
NP = 10240
CH = 128
NCHUNK = 2560
EP = NCHUNK * CH
NT = 16
CPT = NCHUNK // NT
GG = 160
RPT = NP // NT
HR = NP // 16

_mesh = plsc.VectorSubcoreMesh(core_axis_name="c", subcore_axis_name="s")


@functools.partial(
    pl.kernel,
    out_type=jax.ShapeDtypeStruct((2, NP), jnp.float32),
    mesh=_mesh,
    scratch_types=[
        pltpu.VMEM((CPT, CH), jnp.int32),
        pltpu.VMEM((NP,), jnp.float32),
        pltpu.VMEM((NT, RPT), jnp.float32),
        pltpu.VMEM((RPT,), jnp.float32),
        pltpu.VMEM_SHARED((NT, NP), jnp.float32),
    ],
    compiler_params=pltpu.CompilerParams(needs_layout_passes=False),
)
def _deg_kernel(sidx, zeros, deg_out, ixb, hist, rbuf, obuf, part_sh):
    c = lax.axis_index("c")
    s = lax.axis_index("s")
    pltpu.sync_copy(sidx.at[c, pl.ds(s * CPT, CPT)], ixb)
    pltpu.sync_copy(zeros, hist)

    ones = jnp.full((16,), 1.0, jnp.float32)

    def body(j, carry):
        for k in range(8):
            iv = ixb[j, pl.ds(k * 16, 16)]
            plsc.addupdate_scatter(hist, [iv], ones)
        return carry

    lax.fori_loop(0, CPT, body, 0)
    pltpu.sync_copy(hist, part_sh.at[s])
    plsc.subcore_barrier()
    pltpu.sync_copy(part_sh.at[:, pl.ds(s * RPT, RPT)], rbuf)
    for k in range(RPT // 16):
        v = rbuf[0, pl.ds(k * 16, 16)]
        for r in range(1, NT):
            v = v + rbuf[r, pl.ds(k * 16, 16)]
        obuf[pl.ds(k * 16, 16)] = v
    pltpu.sync_copy(obuf, deg_out.at[c, pl.ds(s * RPT, RPT)])


@functools.partial(
    pl.kernel,
    out_type=jax.ShapeDtypeStruct((2, NP, D), jnp.bfloat16),
    mesh=_mesh,
    scratch_types=[
        pltpu.VMEM((GG, CH), jnp.int32),
        pltpu.VMEM((GG, CH), jnp.int32),
        pltpu.VMEM((2, CH, D), jnp.bfloat16),
        pltpu.VMEM_SHARED((NP, D), jnp.bfloat16),
        pltpu.SemaphoreType.DMA,
        pltpu.SemaphoreType.DMA,
        pltpu.SemaphoreType.DMA,
        pltpu.SemaphoreType.DMA,
    ],
    compiler_params=pltpu.CompilerParams(use_tc_tiling_on_sc=False),
)
def _prop_kernel(g, gidx, sidx, out, gix, six, rows, acc, gs0, gs1, ss0, ss1):
    c = lax.axis_index("c")
    s = lax.axis_index("s")
    pltpu.sync_copy(g.at[pl.ds(c * NP + s * RPT, RPT)], acc.at[pl.ds(s * RPT, RPT)])
    plsc.subcore_barrier()

    gsem = (gs0, gs1)
    ssem = (ss0, ss1)

    def g_start(k, b):
        pltpu.make_async_copy(g.at[gix.at[k]], rows.at[b], gsem[b]).start()

    def g_wait(k, b):
        pltpu.make_async_copy(g.at[gix.at[k]], rows.at[b], gsem[b]).wait()

    def s_start(k, b):
        pltpu.async_copy(rows.at[b], acc.at[six.at[k]], ssem[b], add=True)

    def s_wait(k, b):
        pltpu.make_async_copy(rows.at[b], acc.at[six.at[k]], ssem[b]).wait()

    def group(gi, carry):
        base = s * CPT + gi * GG
        pltpu.sync_copy(gidx.at[c, pl.ds(base, GG)], gix)
        pltpu.sync_copy(sidx.at[c, pl.ds(base, GG)], six)
        g_start(0, 0)
        g_start(1, 1)

        def body(j, carry2):
            k0 = 2 * j
            g_wait(k0, 0)
            s_start(k0, 0)
            g_wait(k0 + 1, 1)
            s_start(k0 + 1, 1)
            s_wait(k0, 0)
            g_start(k0 + 2, 0)
            s_wait(k0 + 1, 1)
            g_start(k0 + 3, 1)
            return carry2

        lax.fori_loop(0, GG // 2 - 1, body, 0)
        g_wait(GG - 2, 0)
        s_start(GG - 2, 0)
        g_wait(GG - 1, 1)
        s_start(GG - 1, 1)
        s_wait(GG - 2, 0)
        s_wait(GG - 1, 1)
        return carry

    lax.fori_loop(0, CPT // GG, group, 0)

    plsc.subcore_barrier()
    pltpu.sync_copy(acc.at[pl.ds(s * RPT, RPT)], out.at[c, pl.ds(s * RPT, RPT)])


BN = 2048


def _g_body(x_ref, w_ref, deg_ref, g_ref):
    l = pl.program_id(0)
    dinv = lax.rsqrt(deg_ref[...] + 1.0)
    w = jnp.where(l == 0, dinv[:, 0:1], dinv[:, 1:2])
    g_ref[...] = (
        jnp.dot(x_ref[...], w_ref[0], preferred_element_type=jnp.float32) * w
    ).astype(jnp.bfloat16)


def _g_table(xp, wst, deg_t):
    return pl.pallas_call(
        _g_body,
        grid=(2, NP // BN),
        in_specs=[
            pl.BlockSpec((BN, D), lambda l, i: (i, 0)),
            pl.BlockSpec((1, D, D), lambda l, i: (l, 0, 0)),
            pl.BlockSpec((BN, 2), lambda l, i: (i, 0)),
        ],
        out_specs=pl.BlockSpec((BN, D), lambda l, i: (l * (NP // BN) + i, 0)),
        out_shape=jax.ShapeDtypeStruct((2 * NP, D), jnp.bfloat16),
    )(xp, wst, deg_t)


def _ep_body(acc_ref, deg_ref, b_ref, o_ref):
    dinv = lax.rsqrt(deg_ref[...] + 1.0)
    a0 = acc_ref[0].astype(jnp.float32)
    a1 = acc_ref[1].astype(jnp.float32)
    o1 = jnp.maximum(a0 * dinv[:, 0:1] + b_ref[0][None, :], 0.0)
    o2 = jnp.maximum(a1 * dinv[:, 1:2] + b_ref[1][None, :], 0.0)
    o_ref[...] = 0.5 * (o1 + o2)


def _epilogue(acc, deg_t, bst):
    return pl.pallas_call(
        _ep_body,
        grid=(NP // BN,),
        in_specs=[
            pl.BlockSpec((2, BN, D), lambda i: (0, i, 0)),
            pl.BlockSpec((BN, 2), lambda i: (i, 0)),
            pl.BlockSpec((2, D), lambda i: (0, 0)),
        ],
        out_specs=pl.BlockSpec((BN, D), lambda i: (i, 0)),
        out_shape=jax.ShapeDtypeStruct((NP, D), jnp.float32),
    )(acc, deg_t, bst)


def kernel(x, edge_index, W1, b1, W2, b2):
    src = edge_index[0]
    dst = edge_index[1]
    gidx = jnp.pad(jnp.stack([src, dst + NP]), ((0, 0), (0, EP - E)))
    sidx = jnp.pad(jnp.stack([dst, src]), ((0, 0), (0, EP - E)), constant_values=N)
    gidx = gidx.reshape(2, NCHUNK, CH)
    sidx = sidx.reshape(2, NCHUNK, CH)

    zeros = jnp.zeros((NP,), jnp.float32)
    deg_t = _deg_kernel(sidx, zeros).T

    xp = jnp.pad(x, ((0, NP - N), (0, 0)))
    wst = jnp.stack([W1, W2])
    bst = jnp.stack([b1, b2])
    g = _g_table(xp, wst, deg_t)

    acc = _prop_kernel(g, gidx, sidx)
    out = _epilogue(acc, deg_t, bst)
    return out[:N]


# speedup vs baseline: 1.0429x; 1.0429x over previous
import functools

import jax
import jax.numpy as jnp
from jax import lax
from jax.experimental import pallas as pl
from jax.experimental.pallas import tpu as pltpu
from jax.experimental.pallas import tpu_sc as plsc

N = 10000
E = 320000
D = 128
NP = 10240
CH = 128
NCHUNK = 2560
EP = NCHUNK * CH
NT = 16
CPT = NCHUNK // NT
GG = 160
RPT = NP // NT
HR = NP // 16

_mesh = plsc.VectorSubcoreMesh(core_axis_name="c", subcore_axis_name="s")


@functools.partial(
    pl.kernel,
    out_type=jax.ShapeDtypeStruct((2, NP), jnp.float32),
    mesh=_mesh,
    scratch_types=[
        pltpu.VMEM((CPT, CH), jnp.int32),
        pltpu.VMEM((NP,), jnp.float32),
        pltpu.VMEM((NT, RPT), jnp.float32),
        pltpu.VMEM((RPT,), jnp.float32),
        pltpu.VMEM_SHARED((NT, NP), jnp.float32),
    ],
    compiler_params=pltpu.CompilerParams(needs_layout_passes=False),
)
def _deg_kernel(sidx, zeros, deg_out, ixb, hist, rbuf, obuf, part_sh):
    c = lax.axis_index("c")
    s = lax.axis_index("s")
    pltpu.sync_copy(sidx.at[c, pl.ds(s * CPT, CPT)], ixb)
    pltpu.sync_copy(zeros, hist)

    ones = jnp.full((16,), 1.0, jnp.float32)

    def body(j, carry):
        for k in range(8):
            iv = ixb[j, pl.ds(k * 16, 16)]
            plsc.addupdate_scatter(hist, [iv], ones)
        return carry

    lax.fori_loop(0, CPT, body, 0)
    pltpu.sync_copy(hist, part_sh.at[s])
    plsc.subcore_barrier()
    pltpu.sync_copy(part_sh.at[:, pl.ds(s * RPT, RPT)], rbuf)
    for k in range(RPT // 16):
        v = rbuf[0, pl.ds(k * 16, 16)]
        for r in range(1, NT):
            v = v + rbuf[r, pl.ds(k * 16, 16)]
        obuf[pl.ds(k * 16, 16)] = v
    pltpu.sync_copy(obuf, deg_out.at[c, pl.ds(s * RPT, RPT)])


@functools.partial(
    pl.kernel,
    out_type=jax.ShapeDtypeStruct((2, NP, D), jnp.bfloat16),
    mesh=_mesh,
    scratch_types=[
        pltpu.VMEM((GG, CH), jnp.int32),
        pltpu.VMEM((GG, CH), jnp.int32),
        pltpu.VMEM((2, CH, D), jnp.bfloat16),
        pltpu.VMEM_SHARED((NP, D), jnp.bfloat16),
        pltpu.SemaphoreType.DMA,
        pltpu.SemaphoreType.DMA,
    ],
    compiler_params=pltpu.CompilerParams(use_tc_tiling_on_sc=False),
)
def _prop_kernel(g, gidx, sidx, out, gix, six, rows, acc, gs0, gs1):
    c = lax.axis_index("c")
    s = lax.axis_index("s")
    pltpu.sync_copy(g.at[pl.ds(c * NP + s * RPT, RPT)], acc.at[pl.ds(s * RPT, RPT)])
    plsc.subcore_barrier()

    def group(gi, carry):
        base = s * CPT + gi * GG
        pltpu.sync_copy(gidx.at[c, pl.ds(base, GG)], gix)
        pltpu.sync_copy(sidx.at[c, pl.ds(base, GG)], six)
        pltpu.make_async_copy(g.at[gix.at[0]], rows.at[0], gs0).start()

        def body(j, carry2):
            k0 = 2 * j
            pltpu.make_async_copy(g.at[gix.at[k0 + 1]], rows.at[1], gs1).start()
            pltpu.make_async_copy(g.at[gix.at[k0]], rows.at[0], gs0).wait()
            pltpu.sync_copy(rows.at[0], acc.at[six.at[k0]], add=True)
            pltpu.make_async_copy(g.at[gix.at[k0 + 2]], rows.at[0], gs0).start()
            pltpu.make_async_copy(g.at[gix.at[k0 + 1]], rows.at[1], gs1).wait()
            pltpu.sync_copy(rows.at[1], acc.at[six.at[k0 + 1]], add=True)
            return carry2

        lax.fori_loop(0, GG // 2 - 1, body, 0)
        pltpu.make_async_copy(g.at[gix.at[GG - 1]], rows.at[1], gs1).start()
        pltpu.make_async_copy(g.at[gix.at[GG - 2]], rows.at[0], gs0).wait()
        pltpu.sync_copy(rows.at[0], acc.at[six.at[GG - 2]], add=True)
        pltpu.make_async_copy(g.at[gix.at[GG - 1]], rows.at[1], gs1).wait()
        pltpu.sync_copy(rows.at[1], acc.at[six.at[GG - 1]], add=True)
        return carry

    lax.fori_loop(0, CPT // GG, group, 0)

    plsc.subcore_barrier()
    pltpu.sync_copy(acc.at[pl.ds(s * RPT, RPT)], out.at[c, pl.ds(s * RPT, RPT)])


BN = 2048


def _h_body(x_ref, w_ref, h_ref):
    h_ref[...] = jnp.dot(x_ref[...], w_ref[0], preferred_element_type=jnp.float32)


def _h_table(xp, wst):
    return pl.pallas_call(
        _h_body,
        grid=(2, NP // BN),
        in_specs=[
            pl.BlockSpec((BN, D), lambda l, i: (i, 0)),
            pl.BlockSpec((1, D, D), lambda l, i: (l, 0, 0)),
        ],
        out_specs=pl.BlockSpec((BN, D), lambda l, i: (l * (NP // BN) + i, 0)),
        out_shape=jax.ShapeDtypeStruct((2 * NP, D), jnp.float32),
    )(xp, wst)


def _g_body(h_ref, deg_ref, g_ref):
    l = pl.program_id(0)
    dinv = lax.rsqrt(deg_ref[...] + 1.0)
    w = jnp.where(l == 0, dinv[:, 0:1], dinv[:, 1:2])
    g_ref[...] = (h_ref[...] * w).astype(jnp.bfloat16)


def _g_table(h, deg_t):
    return pl.pallas_call(
        _g_body,
        grid=(2, NP // BN),
        in_specs=[
            pl.BlockSpec((BN, D), lambda l, i: (l * (NP // BN) + i, 0)),
            pl.BlockSpec((BN, 2), lambda l, i: (i, 0)),
        ],
        out_specs=pl.BlockSpec((BN, D), lambda l, i: (l * (NP // BN) + i, 0)),
        out_shape=jax.ShapeDtypeStruct((2 * NP, D), jnp.bfloat16),
    )(h, deg_t)


def _ep_body(acc_ref, deg_ref, b_ref, o_ref):
    dinv = lax.rsqrt(deg_ref[...] + 1.0)
    a0 = acc_ref[0].astype(jnp.float32)
    a1 = acc_ref[1].astype(jnp.float32)
    o1 = jnp.maximum(a0 * dinv[:, 0:1] + b_ref[0][None, :], 0.0)
    o2 = jnp.maximum(a1 * dinv[:, 1:2] + b_ref[1][None, :], 0.0)
    o_ref[...] = 0.5 * (o1 + o2)


def _epilogue(acc, deg_t, bst):
    return pl.pallas_call(
        _ep_body,
        grid=(NP // BN,),
        in_specs=[
            pl.BlockSpec((2, BN, D), lambda i: (0, i, 0)),
            pl.BlockSpec((BN, 2), lambda i: (i, 0)),
            pl.BlockSpec((2, D), lambda i: (0, 0)),
        ],
        out_specs=pl.BlockSpec((BN, D), lambda i: (i, 0)),
        out_shape=jax.ShapeDtypeStruct((NP, D), jnp.float32),
    )(acc, deg_t, bst)


def kernel(x, edge_index, W1, b1, W2, b2):
    src = edge_index[0]
    dst = edge_index[1]
    gidx = jnp.pad(jnp.stack([src, dst + NP]), ((0, 0), (0, EP - E)))
    sidx = jnp.pad(jnp.stack([dst, src]), ((0, 0), (0, EP - E)), constant_values=N)
    gidx = gidx.reshape(2, NCHUNK, CH)
    sidx = sidx.reshape(2, NCHUNK, CH)

    zeros = jnp.zeros((NP,), jnp.float32)
    deg_t = _deg_kernel(sidx, zeros).T

    xp = jnp.pad(x, ((0, NP - N), (0, 0)))
    wst = jnp.stack([W1, W2])
    bst = jnp.stack([b1, b2])
    h = _h_table(xp, wst)
    g = _g_table(h, deg_t)

    acc = _prop_kernel(g, gidx, sidx)
    out = _epilogue(acc, deg_t, bst)
    return out[:N]


# speedup vs baseline: 1.0449x; 1.0019x over previous
import functools

import jax
import jax.numpy as jnp
from jax import lax
from jax.experimental import pallas as pl
from jax.experimental.pallas import tpu as pltpu
from jax.experimental.pallas import tpu_sc as plsc

N = 10000
E = 320000
D = 128
NP = 10240
CH = 128
NCHUNK = 2560
EP = NCHUNK * CH
NT = 16
CPT = NCHUNK // NT
GG = 160
RPT = NP // NT
HR = NP // 16

_mesh = plsc.VectorSubcoreMesh(core_axis_name="c", subcore_axis_name="s")


@functools.partial(
    pl.kernel,
    out_type=jax.ShapeDtypeStruct((2, NP), jnp.float32),
    mesh=_mesh,
    scratch_types=[
        pltpu.VMEM((CPT, CH), jnp.int32),
        pltpu.VMEM((NP,), jnp.float32),
        pltpu.VMEM((NT, RPT), jnp.float32),
        pltpu.VMEM((RPT,), jnp.float32),
        pltpu.VMEM_SHARED((NT, NP), jnp.float32),
    ],
    compiler_params=pltpu.CompilerParams(needs_layout_passes=False),
)
def _deg_kernel(sidx, zeros, deg_out, ixb, hist, rbuf, obuf, part_sh):
    c = lax.axis_index("c")
    s = lax.axis_index("s")
    pltpu.sync_copy(sidx.at[c, pl.ds(s * CPT, CPT)], ixb)
    pltpu.sync_copy(zeros, hist)

    ones = jnp.full((16,), 1.0, jnp.float32)

    def body(j, carry):
        for k in range(16):
            iv = ixb[2 * j + k // 8, pl.ds((k % 8) * 16, 16)]
            plsc.addupdate_scatter(hist, [iv], ones)
        return carry

    lax.fori_loop(0, CPT // 2, body, 0)
    pltpu.sync_copy(hist, part_sh.at[s])
    plsc.subcore_barrier()
    pltpu.sync_copy(part_sh.at[:, pl.ds(s * RPT, RPT)], rbuf)
    for k in range(RPT // 16):
        v = rbuf[0, pl.ds(k * 16, 16)]
        for r in range(1, NT):
            v = v + rbuf[r, pl.ds(k * 16, 16)]
        obuf[pl.ds(k * 16, 16)] = v
    pltpu.sync_copy(obuf, deg_out.at[c, pl.ds(s * RPT, RPT)])


@functools.partial(
    pl.kernel,
    out_type=jax.ShapeDtypeStruct((2, NP, D), jnp.bfloat16),
    mesh=_mesh,
    scratch_types=[
        pltpu.VMEM((GG, CH), jnp.int32),
        pltpu.VMEM((GG, CH), jnp.int32),
        pltpu.VMEM((2, CH, D), jnp.bfloat16),
        pltpu.VMEM_SHARED((NP, D), jnp.bfloat16),
        pltpu.SemaphoreType.DMA,
        pltpu.SemaphoreType.DMA,
    ],
    compiler_params=pltpu.CompilerParams(use_tc_tiling_on_sc=False),
)
def _prop_kernel(g, gidx, sidx, out, gix, six, rows, acc, gs0, gs1):
    c = lax.axis_index("c")
    s = lax.axis_index("s")
    pltpu.sync_copy(g.at[pl.ds(c * NP + s * RPT, RPT)], acc.at[pl.ds(s * RPT, RPT)])
    plsc.subcore_barrier()

    def group(gi, carry):
        base = s * CPT + gi * GG
        pltpu.sync_copy(gidx.at[c, pl.ds(base, GG)], gix)
        pltpu.sync_copy(sidx.at[c, pl.ds(base, GG)], six)
        pltpu.make_async_copy(g.at[gix.at[0]], rows.at[0], gs0).start()

        def body(j, carry2):
            k0 = 2 * j
            pltpu.make_async_copy(g.at[gix.at[k0 + 1]], rows.at[1], gs1).start()
            pltpu.make_async_copy(g.at[gix.at[k0]], rows.at[0], gs0).wait()
            pltpu.sync_copy(rows.at[0], acc.at[six.at[k0]], add=True)
            pltpu.make_async_copy(g.at[gix.at[k0 + 2]], rows.at[0], gs0).start()
            pltpu.make_async_copy(g.at[gix.at[k0 + 1]], rows.at[1], gs1).wait()
            pltpu.sync_copy(rows.at[1], acc.at[six.at[k0 + 1]], add=True)
            return carry2

        lax.fori_loop(0, GG // 2 - 1, body, 0)
        pltpu.make_async_copy(g.at[gix.at[GG - 1]], rows.at[1], gs1).start()
        pltpu.make_async_copy(g.at[gix.at[GG - 2]], rows.at[0], gs0).wait()
        pltpu.sync_copy(rows.at[0], acc.at[six.at[GG - 2]], add=True)
        pltpu.make_async_copy(g.at[gix.at[GG - 1]], rows.at[1], gs1).wait()
        pltpu.sync_copy(rows.at[1], acc.at[six.at[GG - 1]], add=True)
        return carry

    lax.fori_loop(0, CPT // GG, group, 0)

    plsc.subcore_barrier()
    pltpu.sync_copy(acc.at[pl.ds(s * RPT, RPT)], out.at[c, pl.ds(s * RPT, RPT)])


BN = 2048


def _g_body(x_ref, w_ref, deg_ref, g_ref):
    l = pl.program_id(0)
    dinv = lax.rsqrt(deg_ref[...] + 1.0)
    w = jnp.where(l == 0, dinv[:, 0:1], dinv[:, 1:2])
    g_ref[...] = (
        jnp.dot(x_ref[...], w_ref[0], preferred_element_type=jnp.float32) * w
    ).astype(jnp.bfloat16)


def _g_table(xp, wst, deg_t):
    return pl.pallas_call(
        _g_body,
        grid=(2, NP // BN),
        in_specs=[
            pl.BlockSpec((BN, D), lambda l, i: (i, 0)),
            pl.BlockSpec((1, D, D), lambda l, i: (l, 0, 0)),
            pl.BlockSpec((BN, 2), lambda l, i: (i, 0)),
        ],
        out_specs=pl.BlockSpec((BN, D), lambda l, i: (l * (NP // BN) + i, 0)),
        out_shape=jax.ShapeDtypeStruct((2 * NP, D), jnp.bfloat16),
    )(xp, wst, deg_t)


def _ep_body(acc_ref, deg_ref, b_ref, o_ref):
    dinv = lax.rsqrt(deg_ref[...] + 1.0)
    a0 = acc_ref[0].astype(jnp.float32)
    a1 = acc_ref[1].astype(jnp.float32)
    o1 = jnp.maximum(a0 * dinv[:, 0:1] + b_ref[0][None, :], 0.0)
    o2 = jnp.maximum(a1 * dinv[:, 1:2] + b_ref[1][None, :], 0.0)
    o_ref[...] = 0.5 * (o1 + o2)


def _epilogue(acc, deg_t, bst):
    return pl.pallas_call(
        _ep_body,
        grid=(NP // BN,),
        in_specs=[
            pl.BlockSpec((2, BN, D), lambda i: (0, i, 0)),
            pl.BlockSpec((BN, 2), lambda i: (i, 0)),
            pl.BlockSpec((2, D), lambda i: (0, 0)),
        ],
        out_specs=pl.BlockSpec((BN, D), lambda i: (i, 0)),
        out_shape=jax.ShapeDtypeStruct((NP, D), jnp.float32),
    )(acc, deg_t, bst)


def kernel(x, edge_index, W1, b1, W2, b2):
    src = edge_index[0]
    dst = edge_index[1]
    gidx = jnp.pad(jnp.stack([src, dst + NP]), ((0, 0), (0, EP - E)))
    sidx = jnp.pad(jnp.stack([dst, src]), ((0, 0), (0, EP - E)), constant_values=N)
    gidx = gidx.reshape(2, NCHUNK, CH)
    sidx = sidx.reshape(2, NCHUNK, CH)

    zeros = jnp.zeros((NP,), jnp.float32)
    deg_t = _deg_kernel(sidx, zeros).T

    xp = jnp.pad(x, ((0, NP - N), (0, 0)))
    wst = jnp.stack([W1, W2])
    bst = jnp.stack([b1, b2])
    g = _g_table(xp, wst, deg_t)

    acc = _prop_kernel(g, gidx, sidx)
    out = _epilogue(acc, deg_t, bst)
    return out[:N]


# speedup vs baseline: 1.0462x; 1.0013x over previous
import functools

import jax
import jax.numpy as jnp
from jax import lax
from jax.experimental import pallas as pl
from jax.experimental.pallas import tpu as pltpu
from jax.experimental.pallas import tpu_sc as plsc

N = 10000
E = 320000
D = 128
NP = 10240
CH = 128
NCHUNK = 2560
EP = NCHUNK * CH
NT = 16
CPT = NCHUNK // NT
GG = 160
RPT = NP // NT

_mesh = plsc.VectorSubcoreMesh(core_axis_name="c", subcore_axis_name="s")


@functools.partial(
    pl.kernel,
    out_type=jax.ShapeDtypeStruct((2, NP), jnp.float32),
    mesh=_mesh,
    scratch_types=[
        pltpu.VMEM((CPT, CH), jnp.int32),
        pltpu.VMEM((NP,), jnp.float32),
        pltpu.VMEM((NT, RPT), jnp.float32),
        pltpu.VMEM((RPT,), jnp.float32),
        pltpu.VMEM_SHARED((NT, NP), jnp.float32),
    ],
    compiler_params=pltpu.CompilerParams(needs_layout_passes=False),
)
def _deg_kernel(sidx, zeros, deg_out, ixb, hist, rbuf, obuf, part_sh):
    c = lax.axis_index("c")
    s = lax.axis_index("s")
    pltpu.sync_copy(sidx.at[c, pl.ds(s * CPT, CPT)], ixb)
    pltpu.sync_copy(zeros, hist)

    ones = jnp.full((16,), 1.0, jnp.float32)

    def body(j, carry):
        for k in range(16):
            iv = ixb[2 * j + k // 8, pl.ds((k % 8) * 16, 16)]
            plsc.addupdate_scatter(hist, [iv], ones)
        return carry

    lax.fori_loop(0, CPT // 2, body, 0)
    pltpu.sync_copy(hist, part_sh.at[s])
    plsc.subcore_barrier()
    pltpu.sync_copy(part_sh.at[:, pl.ds(s * RPT, RPT)], rbuf)
    for k in range(RPT // 16):
        v = rbuf[0, pl.ds(k * 16, 16)]
        for r in range(1, NT):
            v = v + rbuf[r, pl.ds(k * 16, 16)]
        obuf[pl.ds(k * 16, 16)] = v
    pltpu.sync_copy(obuf, deg_out.at[c, pl.ds(s * RPT, RPT)])


@functools.partial(
    pl.kernel,
    out_type=jax.ShapeDtypeStruct((2, NP, D), jnp.bfloat16),
    mesh=_mesh,
    scratch_types=[
        pltpu.VMEM((GG, CH), jnp.int32),
        pltpu.VMEM((GG, CH), jnp.int32),
        pltpu.VMEM((2, CH, D), jnp.bfloat16),
        pltpu.VMEM_SHARED((NP, D), jnp.bfloat16),
        pltpu.SemaphoreType.DMA,
        pltpu.SemaphoreType.DMA,
    ],
    compiler_params=pltpu.CompilerParams(use_tc_tiling_on_sc=False),
)
def _prop_kernel(g, gidx, sidx, out, gix, six, rows, acc, gs0, gs1):
    c = lax.axis_index("c")
    s = lax.axis_index("s")
    pltpu.sync_copy(g.at[pl.ds(c * NP + s * RPT, RPT)], acc.at[pl.ds(s * RPT, RPT)])
    plsc.subcore_barrier()

    def group(gi, carry):
        base = s * CPT + gi * GG
        pltpu.sync_copy(gidx.at[c, pl.ds(base, GG)], gix)
        pltpu.sync_copy(sidx.at[c, pl.ds(base, GG)], six)
        pltpu.make_async_copy(g.at[gix.at[0]], rows.at[0], gs0).start()

        def body(j, carry2):
            k0 = 2 * j
            pltpu.make_async_copy(g.at[gix.at[k0 + 1]], rows.at[1], gs1).start()
            pltpu.make_async_copy(g.at[gix.at[k0]], rows.at[0], gs0).wait()
            pltpu.sync_copy(rows.at[0], acc.at[six.at[k0]], add=True)
            pltpu.make_async_copy(g.at[gix.at[k0 + 2]], rows.at[0], gs0).start()
            pltpu.make_async_copy(g.at[gix.at[k0 + 1]], rows.at[1], gs1).wait()
            pltpu.sync_copy(rows.at[1], acc.at[six.at[k0 + 1]], add=True)
            return carry2

        lax.fori_loop(0, GG // 2 - 1, body, 0)
        pltpu.make_async_copy(g.at[gix.at[GG - 1]], rows.at[1], gs1).start()
        pltpu.make_async_copy(g.at[gix.at[GG - 2]], rows.at[0], gs0).wait()
        pltpu.sync_copy(rows.at[0], acc.at[six.at[GG - 2]], add=True)
        pltpu.make_async_copy(g.at[gix.at[GG - 1]], rows.at[1], gs1).wait()
        pltpu.sync_copy(rows.at[1], acc.at[six.at[GG - 1]], add=True)
        return carry

    lax.fori_loop(0, CPT // GG, group, 0)

    plsc.subcore_barrier()
    pltpu.sync_copy(acc.at[pl.ds(s * RPT, RPT)], out.at[c, pl.ds(s * RPT, RPT)])


BN = 2048


def _g_body(x_ref, w_ref, deg_ref, g_ref):
    l = pl.program_id(0)
    dinv = lax.rsqrt(deg_ref[...] + 1.0)
    w = jnp.where(l == 0, dinv[:, 0:1], dinv[:, 1:2])
    g_ref[...] = (
        jnp.dot(x_ref[...], w_ref[0], preferred_element_type=jnp.float32) * w
    ).astype(jnp.bfloat16)


def _g_table(xp, wst, deg_t):
    return pl.pallas_call(
        _g_body,
        grid=(2, NP // BN),
        in_specs=[
            pl.BlockSpec((BN, D), lambda l, i: (i, 0)),
            pl.BlockSpec((1, D, D), lambda l, i: (l, 0, 0)),
            pl.BlockSpec((BN, 2), lambda l, i: (i, 0)),
        ],
        out_specs=pl.BlockSpec((BN, D), lambda l, i: (l * (NP // BN) + i, 0)),
        out_shape=jax.ShapeDtypeStruct((2 * NP, D), jnp.bfloat16),
    )(xp, wst, deg_t)


def _ep_body(acc_ref, deg_ref, b_ref, o_ref):
    dinv = lax.rsqrt(deg_ref[...] + 1.0)
    a0 = acc_ref[0].astype(jnp.float32)
    a1 = acc_ref[1].astype(jnp.float32)
    o1 = jnp.maximum(a0 * dinv[:, 0:1] + b_ref[0][None, :], 0.0)
    o2 = jnp.maximum(a1 * dinv[:, 1:2] + b_ref[1][None, :], 0.0)
    o_ref[...] = 0.5 * (o1 + o2)


def _epilogue(acc, deg_t, bst):
    return pl.pallas_call(
        _ep_body,
        grid=(NP // BN,),
        in_specs=[
            pl.BlockSpec((2, BN, D), lambda i: (0, i, 0)),
            pl.BlockSpec((BN, 2), lambda i: (i, 0)),
            pl.BlockSpec((2, D), lambda i: (0, 0)),
        ],
        out_specs=pl.BlockSpec((BN, D), lambda i: (i, 0)),
        out_shape=jax.ShapeDtypeStruct((NP, D), jnp.float32),
    )(acc, deg_t, bst)


def kernel(x, edge_index, W1, b1, W2, b2):
    src = edge_index[0]
    dst = edge_index[1]
    gidx = jnp.pad(jnp.stack([src, dst + NP]), ((0, 0), (0, EP - E)))
    sidx = jnp.pad(jnp.stack([dst, src]), ((0, 0), (0, EP - E)), constant_values=N)
    gidx = gidx.reshape(2, NCHUNK, CH)
    sidx = sidx.reshape(2, NCHUNK, CH)

    zeros = jnp.zeros((NP,), jnp.float32)
    deg_t = _deg_kernel(sidx, zeros).T

    xp = jnp.pad(x, ((0, NP - N), (0, 0)))
    wst = jnp.stack([W1, W2])
    bst = jnp.stack([b1, b2])
    g = _g_table(xp, wst, deg_t)

    acc = _prop_kernel(g, gidx, sidx)
    out = _epilogue(acc, deg_t, bst)
    return out[:N]
